# Initial kernel scaffold; baseline (speedup 1.0000x reference)
import jax
import jax.numpy as jnp
from jax.experimental import pallas as pl


def kernel(x, sim_weight, dist_weight, conv_w, bn_gamma, bn_beta):
    raise NotImplementedError("write your pallas kernel here")


# speedup vs baseline: 1.3376x; 1.3376x over previous
import jax
import jax.numpy as jnp
from jax import lax
from jax.experimental import pallas as pl
from jax.experimental.pallas import tpu as pltpu

_B, _C, _H, _W = 8, 128, 64, 64
_N = _H * _W
_OUT = 128
_JB = 256
_IB = 256
_BIG = 1 << 30


def _fill_spatial(sp_ref, half, width, dw, blk):
    nblk = half // blk

    def body(k, _):
        i0 = pl.multiple_of(k * blk, blk)
        ii = lax.broadcasted_iota(jnp.int32, (blk, half), 0) + i0
        jj = lax.broadcasted_iota(jnp.int32, (blk, half), 1) + half
        dr = ii // width - jj // width
        dc = ii % width - jj % width
        d2 = (dr * dr + dc * dc).astype(jnp.float32)
        sp_ref[pl.ds(i0, blk), :] = dw * (1.0 / (jnp.sqrt(d2) + 1e-6))
        return 0

    lax.fori_loop(0, nblk, body, 0, unroll=False)


def _round(get_a, get_b, sp_ref, dst_ref, half, sw):
    xa = get_a(0, half)
    na2 = jnp.sum(xa * xa, axis=0, keepdims=True)
    ma = (xa / jnp.sqrt(na2)).astype(jnp.bfloat16)

    nj = half // _JB

    def jbody(t, carry):
        best, bestj = carry
        j0 = pl.multiple_of(t * _JB, _JB)
        xbt = get_b(j0, _JB)
        nb2 = jnp.sum(xbt * xbt, axis=0, keepdims=True)
        mb = (xbt / jnp.sqrt(nb2)).astype(jnp.bfloat16)
        mm = lax.dot_general(ma, mb, (((0,), (0,)), ((), ())),
                             preferred_element_type=jnp.float32)
        s = sw * mm + sp_ref[:, pl.ds(j0, _JB)]
        tmax = jnp.max(s, axis=1, keepdims=True)
        jio = lax.broadcasted_iota(jnp.int32, (half, _JB), 1) + j0
        targ = jnp.min(jnp.where(s == tmax, jio, _BIG), axis=1,
                       keepdims=True)
        upd = tmax > best
        return jnp.where(upd, tmax, best), jnp.where(upd, targ, bestj)

    best0 = jnp.full((half, 1), -jnp.inf, jnp.float32)
    bestj0 = jnp.zeros((half, 1), jnp.int32)
    _, dst = lax.fori_loop(0, nj, jbody, (best0, bestj0), unroll=False)
    dst_ref[pl.ds(0, half), :] = dst

    ni = half // _IB

    def ibody(t, carry):
        sums, cnt = carry
        i0 = pl.multiple_of(t * _IB, _IB)
        xat = get_a(i0, _IB)
        dst_t = dst_ref[pl.ds(i0, _IB), :]
        jio = lax.broadcasted_iota(jnp.int32, (_IB, half), 1)
        oh = (dst_t == jio).astype(jnp.float32)
        sums = sums + lax.dot_general(
            xat, oh, (((1,), (0,)), ((), ())),
            precision=lax.Precision.HIGHEST,
            preferred_element_type=jnp.float32)
        cnt = cnt + lax.dot_general(
            jnp.ones((1, _IB), jnp.float32), oh, (((1,), (0,)), ((), ())),
            precision=lax.Precision.HIGHEST,
            preferred_element_type=jnp.float32)
        return sums, cnt

    sums0 = jnp.zeros((_C, half), jnp.float32)
    cnt0 = jnp.zeros((1, half), jnp.float32)
    sums, cnt = lax.fori_loop(0, ni, ibody, (sums0, cnt0), unroll=False)

    xb = get_b(0, half)
    return (xb + sums) / (1.0 + cnt)


def _body(sw_ref, dw_ref, x_ref, w_ref, g_ref, be_ref, out_ref,
          sp1_ref, sp2_ref, m_ref, dst_ref):
    b = pl.program_id(0)
    sw = sw_ref[0]
    dw = dw_ref[0]

    @pl.when(b == 0)
    def _():
        _fill_spatial(sp1_ref, _N // 2, _W, dw, 256)
        _fill_spatial(sp2_ref, _N // 4, 45, dw, 256)

    h1 = _N // 2

    def a1(c0, w):
        return x_ref[0, :, pl.ds(c0, w)]

    def b1(c0, w):
        return x_ref[0, :, pl.ds(pl.multiple_of(h1 + c0, _JB), w)]

    merged1 = _round(a1, b1, sp1_ref, dst_ref, h1, sw)
    m_ref[:, :] = merged1

    h2 = _N // 4

    def a2(c0, w):
        return m_ref[:, pl.ds(c0, w)]

    def b2(c0, w):
        return m_ref[:, pl.ds(pl.multiple_of(h2 + c0, _JB), w)]

    merged2 = _round(a2, b2, sp2_ref, dst_ref, h2, sw)

    y = lax.dot_general(w_ref[:, :].astype(jnp.bfloat16),
                        merged2.astype(jnp.bfloat16),
                        (((1,), (0,)), ((), ())),
                        preferred_element_type=jnp.float32)
    scale = g_ref[:, :] / jnp.sqrt(jnp.float32(1.0 + 1e-3))
    y = y * scale + be_ref[:, :]
    out_ref[0, :, :] = y * (1.0 / (1.0 + jnp.exp(-y)))


def kernel(x, sim_weight, dist_weight, conv_w, bn_gamma, bn_beta):
    xr = x.reshape(_B, _C, _N)
    sw = jnp.reshape(sim_weight, (1,)).astype(jnp.float32)
    dw = jnp.reshape(dist_weight, (1,)).astype(jnp.float32)
    g = jnp.reshape(bn_gamma, (_OUT, 1)).astype(jnp.float32)
    be = jnp.reshape(bn_beta, (_OUT, 1)).astype(jnp.float32)

    out = pl.pallas_call(
        _body,
        grid=(_B,),
        in_specs=[
            pl.BlockSpec(memory_space=pltpu.SMEM),
            pl.BlockSpec(memory_space=pltpu.SMEM),
            pl.BlockSpec((1, _C, _N), lambda b: (b, 0, 0)),
            pl.BlockSpec((_OUT, _C), lambda b: (0, 0)),
            pl.BlockSpec((_OUT, 1), lambda b: (0, 0)),
            pl.BlockSpec((_OUT, 1), lambda b: (0, 0)),
        ],
        out_specs=pl.BlockSpec((1, _OUT, _N // 4), lambda b: (b, 0, 0)),
        out_shape=jax.ShapeDtypeStruct((_B, _OUT, _N // 4), jnp.float32),
        scratch_shapes=[
            pltpu.VMEM((_N // 2, _N // 2), jnp.float32),
            pltpu.VMEM((_N // 4, _N // 4), jnp.float32),
            pltpu.VMEM((_C, _N // 2), jnp.float32),
            pltpu.VMEM((_N // 2, 1), jnp.int32),
        ],
        compiler_params=pltpu.CompilerParams(
            dimension_semantics=("arbitrary",)),
    )(sw, dw, xr, conv_w, g, be)
    return out.reshape(_B, _OUT, _H // 2, _W // 2)


# speedup vs baseline: 1.8782x; 1.4042x over previous
import jax
import jax.numpy as jnp
from jax import lax
from jax.experimental import pallas as pl
from jax.experimental.pallas import tpu as pltpu

_B, _C, _H, _W = 8, 128, 64, 64
_N = _H * _W
_OUT = 128
_JB = 256
_IB = 256
_BIG = 1 << 30


def _fill_spatial(sp_ref, half, width, dw, blk):
    nblk = half // blk

    def body(k, _):
        i0 = pl.multiple_of(k * blk, blk)
        ii = lax.broadcasted_iota(jnp.int32, (blk, 1), 0) + i0
        jj = lax.broadcasted_iota(jnp.int32, (1, half), 1) + half
        ra = (ii // width).astype(jnp.float32)
        ca = (ii % width).astype(jnp.float32)
        rb = (jj // width).astype(jnp.float32)
        cb = (jj % width).astype(jnp.float32)
        dr = ra - rb
        dc = ca - cb
        d2 = dr * dr + dc * dc
        sp_ref[pl.ds(i0, blk), :] = dw * (1.0 / (jnp.sqrt(d2) + 1e-6))
        return 0

    lax.fori_loop(0, nblk, body, 0, unroll=False)


def _round(get_a, get_b, sp_ref, dst_ref, half, sw):
    xa = get_a(0, half)
    na2 = jnp.sum(xa * xa, axis=0, keepdims=True)
    ma = (xa / jnp.sqrt(na2)).astype(jnp.bfloat16)

    nj = half // _JB

    def jbody(t, carry):
        best, bestj = carry
        j0 = pl.multiple_of(t * _JB, _JB)
        xbt = get_b(j0, _JB)
        nb2 = jnp.sum(xbt * xbt, axis=0, keepdims=True)
        mb = (xbt / jnp.sqrt(nb2)).astype(jnp.bfloat16)
        mm = lax.dot_general(ma, mb, (((0,), (0,)), ((), ())),
                             preferred_element_type=jnp.float32)
        s = sw * mm + sp_ref[:, pl.ds(j0, _JB)]
        tmax = jnp.max(s, axis=1, keepdims=True)
        jio = lax.broadcasted_iota(jnp.int32, (half, _JB), 1) + j0
        targ = jnp.min(jnp.where(s == tmax, jio, _BIG), axis=1,
                       keepdims=True)
        upd = tmax > best
        return jnp.where(upd, tmax, best), jnp.where(upd, targ, bestj)

    best0 = jnp.full((half, 1), -jnp.inf, jnp.float32)
    bestj0 = jnp.zeros((half, 1), jnp.int32)
    _, dst = lax.fori_loop(0, nj, jbody, (best0, bestj0), unroll=False)
    dst_ref[pl.ds(0, half), :] = dst

    ni = half // _IB

    def ibody(t, carry):
        sums, cnt = carry
        i0 = pl.multiple_of(t * _IB, _IB)
        xat = get_a(i0, _IB)
        dst_t = dst_ref[pl.ds(i0, _IB), :]
        jio = lax.broadcasted_iota(jnp.int32, (_IB, half), 1)
        oh = (dst_t == jio).astype(jnp.bfloat16)
        hi = xat.astype(jnp.bfloat16)
        r1 = xat - hi.astype(jnp.float32)
        lo = r1.astype(jnp.bfloat16)
        lo2 = (r1 - lo.astype(jnp.float32)).astype(jnp.bfloat16)
        dn = (((1,), (0,)), ((), ()))
        part = (lax.dot_general(hi, oh, dn, preferred_element_type=jnp.float32)
                + lax.dot_general(lo, oh, dn, preferred_element_type=jnp.float32)
                + lax.dot_general(lo2, oh, dn, preferred_element_type=jnp.float32))
        sums = sums + part
        cnt = cnt + lax.dot_general(
            jnp.ones((1, _IB), jnp.bfloat16), oh, dn,
            preferred_element_type=jnp.float32)
        return sums, cnt

    sums0 = jnp.zeros((_C, half), jnp.float32)
    cnt0 = jnp.zeros((1, half), jnp.float32)
    sums, cnt = lax.fori_loop(0, ni, ibody, (sums0, cnt0), unroll=False)

    xb = get_b(0, half)
    return (xb + sums) / (1.0 + cnt)


def _body(sw_ref, dw_ref, x_ref, w_ref, g_ref, be_ref, out_ref,
          sp1_ref, sp2_ref, m_ref, dst_ref):
    b = pl.program_id(0)
    sw = sw_ref[0]
    dw = dw_ref[0]

    @pl.when(b == 0)
    def _():
        _fill_spatial(sp1_ref, _N // 2, _W, dw, 256)
        _fill_spatial(sp2_ref, _N // 4, 45, dw, 256)

    h1 = _N // 2

    def a1(c0, w):
        return x_ref[0, :, pl.ds(c0, w)]

    def b1(c0, w):
        return x_ref[0, :, pl.ds(pl.multiple_of(h1 + c0, _JB), w)]

    merged1 = _round(a1, b1, sp1_ref, dst_ref, h1, sw)
    m_ref[:, :] = merged1

    h2 = _N // 4

    def a2(c0, w):
        return m_ref[:, pl.ds(c0, w)]

    def b2(c0, w):
        return m_ref[:, pl.ds(pl.multiple_of(h2 + c0, _JB), w)]

    merged2 = _round(a2, b2, sp2_ref, dst_ref, h2, sw)

    y = lax.dot_general(w_ref[:, :].astype(jnp.bfloat16),
                        merged2.astype(jnp.bfloat16),
                        (((1,), (0,)), ((), ())),
                        preferred_element_type=jnp.float32)
    scale = g_ref[:, :] / jnp.sqrt(jnp.float32(1.0 + 1e-3))
    y = y * scale + be_ref[:, :]
    out_ref[0, :, :] = y * (1.0 / (1.0 + jnp.exp(-y)))


def kernel(x, sim_weight, dist_weight, conv_w, bn_gamma, bn_beta):
    xr = x.reshape(_B, _C, _N)
    sw = jnp.reshape(sim_weight, (1,)).astype(jnp.float32)
    dw = jnp.reshape(dist_weight, (1,)).astype(jnp.float32)
    g = jnp.reshape(bn_gamma, (_OUT, 1)).astype(jnp.float32)
    be = jnp.reshape(bn_beta, (_OUT, 1)).astype(jnp.float32)

    out = pl.pallas_call(
        _body,
        grid=(_B,),
        in_specs=[
            pl.BlockSpec(memory_space=pltpu.SMEM),
            pl.BlockSpec(memory_space=pltpu.SMEM),
            pl.BlockSpec((1, _C, _N), lambda b: (b, 0, 0)),
            pl.BlockSpec((_OUT, _C), lambda b: (0, 0)),
            pl.BlockSpec((_OUT, 1), lambda b: (0, 0)),
            pl.BlockSpec((_OUT, 1), lambda b: (0, 0)),
        ],
        out_specs=pl.BlockSpec((1, _OUT, _N // 4), lambda b: (b, 0, 0)),
        out_shape=jax.ShapeDtypeStruct((_B, _OUT, _N // 4), jnp.float32),
        scratch_shapes=[
            pltpu.VMEM((_N // 2, _N // 2), jnp.float32),
            pltpu.VMEM((_N // 4, _N // 4), jnp.float32),
            pltpu.VMEM((_C, _N // 2), jnp.float32),
            pltpu.VMEM((_N // 2, 1), jnp.int32),
        ],
        compiler_params=pltpu.CompilerParams(
            dimension_semantics=("arbitrary",)),
    )(sw, dw, xr, conv_w, g, be)
    return out.reshape(_B, _OUT, _H // 2, _W // 2)


# speedup vs baseline: 2.6172x; 1.3934x over previous
import jax
import jax.numpy as jnp
from jax import lax
from jax.experimental import pallas as pl
from jax.experimental.pallas import tpu as pltpu

_B, _C, _H, _W = 8, 128, 64, 64
_N = _H * _W
_OUT = 128
_JB = 512
_IB = 512
_BIG = 1 << 30


def _fill_spatial(sp_ref, half, width, dw, blk):
    nblk = half // blk

    def body(k, _):
        i0 = pl.multiple_of(k * blk, blk)
        ii = lax.broadcasted_iota(jnp.int32, (blk, 1), 0) + i0
        jj = lax.broadcasted_iota(jnp.int32, (1, half), 1) + half
        ra = (ii // width).astype(jnp.float32)
        ca = (ii % width).astype(jnp.float32)
        rb = (jj // width).astype(jnp.float32)
        cb = (jj % width).astype(jnp.float32)
        dr = ra - rb
        dc = ca - cb
        d2 = dr * dr + dc * dc
        sp_ref[pl.ds(i0, blk), :] = dw * (1.0 / (jnp.sqrt(d2) + 1e-6))
        return 0

    lax.fori_loop(0, nblk, body, 0, unroll=False)


def _round(get_a, get_b, sp_ref, dst_ref, half, sw):
    xa = get_a(0, half)
    na2 = jnp.sum(xa * xa, axis=0, keepdims=True)
    ma = (xa / jnp.sqrt(na2)).astype(jnp.bfloat16)

    nj = half // _JB

    def jbody(t, carry):
        best, bestj = carry
        j0 = pl.multiple_of(t * _JB, _JB)
        xbt = get_b(j0, _JB)
        nb2 = jnp.sum(xbt * xbt, axis=0, keepdims=True)
        mb = (xbt / jnp.sqrt(nb2)).astype(jnp.bfloat16)
        mm = lax.dot_general(ma, mb, (((0,), (0,)), ((), ())),
                             preferred_element_type=jnp.float32)
        s = sw * mm + sp_ref[:, pl.ds(j0, _JB)]
        tmax = jnp.max(s, axis=1, keepdims=True)
        jio = lax.broadcasted_iota(jnp.int32, (half, _JB), 1) + j0
        targ = jnp.min(jnp.where(s == tmax, jio, _BIG), axis=1,
                       keepdims=True)
        upd = tmax > best
        return jnp.where(upd, tmax, best), jnp.where(upd, targ, bestj)

    best0 = jnp.full((half, 1), -jnp.inf, jnp.float32)
    bestj0 = jnp.zeros((half, 1), jnp.int32)
    _, dst = lax.fori_loop(0, nj, jbody, (best0, bestj0), unroll=False)
    dst_ref[pl.ds(0, half), :] = dst

    ni = half // _IB

    def ibody(t, carry):
        sums, cnt = carry
        i0 = pl.multiple_of(t * _IB, _IB)
        xat = get_a(i0, _IB)
        dst_t = dst_ref[pl.ds(i0, _IB), :]
        jio = lax.broadcasted_iota(jnp.int32, (_IB, half), 1)
        oh = (dst_t == jio).astype(jnp.bfloat16)
        hi = xat.astype(jnp.bfloat16)
        r1 = xat - hi.astype(jnp.float32)
        lo = r1.astype(jnp.bfloat16)
        lo2 = (r1 - lo.astype(jnp.float32)).astype(jnp.bfloat16)
        dn = (((1,), (0,)), ((), ()))
        part = (lax.dot_general(hi, oh, dn, preferred_element_type=jnp.float32)
                + lax.dot_general(lo, oh, dn, preferred_element_type=jnp.float32)
                + lax.dot_general(lo2, oh, dn, preferred_element_type=jnp.float32))
        sums = sums + part
        cnt = cnt + lax.dot_general(
            jnp.ones((1, _IB), jnp.bfloat16), oh, dn,
            preferred_element_type=jnp.float32)
        return sums, cnt

    sums0 = jnp.zeros((_C, half), jnp.float32)
    cnt0 = jnp.zeros((1, half), jnp.float32)
    sums, cnt = lax.fori_loop(0, ni, ibody, (sums0, cnt0), unroll=False)

    xb = get_b(0, half)
    return (xb + sums) / (1.0 + cnt)


def _body(sw_ref, dw_ref, x_ref, w_ref, g_ref, be_ref, out_ref,
          sp1_ref, sp2_ref, m_ref, dst_ref):
    b = pl.program_id(0)
    sw = sw_ref[0]
    dw = dw_ref[0]

    @pl.when(b == 0)
    def _():
        _fill_spatial(sp1_ref, _N // 2, _W, dw, 256)
        _fill_spatial(sp2_ref, _N // 4, 45, dw, 256)

    h1 = _N // 2

    def a1(c0, w):
        return x_ref[0, :, pl.ds(c0, w)]

    def b1(c0, w):
        return x_ref[0, :, pl.ds(pl.multiple_of(h1 + c0, _JB), w)]

    merged1 = _round(a1, b1, sp1_ref, dst_ref, h1, sw)
    m_ref[:, :] = merged1

    h2 = _N // 4

    def a2(c0, w):
        return m_ref[:, pl.ds(c0, w)]

    def b2(c0, w):
        return m_ref[:, pl.ds(pl.multiple_of(h2 + c0, _JB), w)]

    merged2 = _round(a2, b2, sp2_ref, dst_ref, h2, sw)

    y = lax.dot_general(w_ref[:, :].astype(jnp.bfloat16),
                        merged2.astype(jnp.bfloat16),
                        (((1,), (0,)), ((), ())),
                        preferred_element_type=jnp.float32)
    scale = g_ref[:, :] / jnp.sqrt(jnp.float32(1.0 + 1e-3))
    y = y * scale + be_ref[:, :]
    out_ref[0, :, :] = y * (1.0 / (1.0 + jnp.exp(-y)))


def kernel(x, sim_weight, dist_weight, conv_w, bn_gamma, bn_beta):
    xr = x.reshape(_B, _C, _N)
    sw = jnp.reshape(sim_weight, (1,)).astype(jnp.float32)
    dw = jnp.reshape(dist_weight, (1,)).astype(jnp.float32)
    g = jnp.reshape(bn_gamma, (_OUT, 1)).astype(jnp.float32)
    be = jnp.reshape(bn_beta, (_OUT, 1)).astype(jnp.float32)

    out = pl.pallas_call(
        _body,
        grid=(_B,),
        in_specs=[
            pl.BlockSpec(memory_space=pltpu.SMEM),
            pl.BlockSpec(memory_space=pltpu.SMEM),
            pl.BlockSpec((1, _C, _N), lambda b: (b, 0, 0)),
            pl.BlockSpec((_OUT, _C), lambda b: (0, 0)),
            pl.BlockSpec((_OUT, 1), lambda b: (0, 0)),
            pl.BlockSpec((_OUT, 1), lambda b: (0, 0)),
        ],
        out_specs=pl.BlockSpec((1, _OUT, _N // 4), lambda b: (b, 0, 0)),
        out_shape=jax.ShapeDtypeStruct((_B, _OUT, _N // 4), jnp.float32),
        scratch_shapes=[
            pltpu.VMEM((_N // 2, _N // 2), jnp.float32),
            pltpu.VMEM((_N // 4, _N // 4), jnp.float32),
            pltpu.VMEM((_C, _N // 2), jnp.float32),
            pltpu.VMEM((_N // 2, 1), jnp.int32),
        ],
        compiler_params=pltpu.CompilerParams(
            dimension_semantics=("arbitrary",)),
    )(sw, dw, xr, conv_w, g, be)
    return out.reshape(_B, _OUT, _H // 2, _W // 2)


# speedup vs baseline: 3.0364x; 1.1602x over previous
import jax
import jax.numpy as jnp
from jax import lax
from jax.experimental import pallas as pl
from jax.experimental.pallas import tpu as pltpu

_B, _C, _H, _W = 8, 128, 64, 64
_N = _H * _W
_OUT = 128
_JB = 1024
_IB = 1024
_BIG = 1 << 30


def _fill_spatial(sp_ref, half, width, dw, blk):
    nblk = half // blk

    def body(k, _):
        i0 = pl.multiple_of(k * blk, blk)
        ii = lax.broadcasted_iota(jnp.int32, (blk, 1), 0) + i0
        jj = lax.broadcasted_iota(jnp.int32, (1, half), 1) + half
        ra = (ii // width).astype(jnp.float32)
        ca = (ii % width).astype(jnp.float32)
        rb = (jj // width).astype(jnp.float32)
        cb = (jj % width).astype(jnp.float32)
        dr = ra - rb
        dc = ca - cb
        d2 = dr * dr + dc * dc
        sp_ref[pl.ds(i0, blk), :] = dw * (1.0 / (jnp.sqrt(d2) + 1e-6))
        return 0

    lax.fori_loop(0, nblk, body, 0, unroll=False)


def _round(get_a, get_b, sp_ref, dst_ref, half, sw):
    xa = get_a(0, half)
    na2 = jnp.sum(xa * xa, axis=0, keepdims=True)
    ma = (xa / jnp.sqrt(na2)).astype(jnp.bfloat16)

    nj = half // _JB

    def jbody(t, carry):
        best, bestj = carry
        j0 = pl.multiple_of(t * _JB, _JB)
        xbt = get_b(j0, _JB)
        nb2 = jnp.sum(xbt * xbt, axis=0, keepdims=True)
        mb = (xbt / jnp.sqrt(nb2)).astype(jnp.bfloat16)
        mm = lax.dot_general(ma, mb, (((0,), (0,)), ((), ())),
                             preferred_element_type=jnp.float32)
        s = sw * mm + sp_ref[:, pl.ds(j0, _JB)]
        tmax = jnp.max(s, axis=1, keepdims=True)
        jio = lax.broadcasted_iota(jnp.int32, (half, _JB), 1) + j0
        targ = jnp.min(jnp.where(s == tmax, jio, _BIG), axis=1,
                       keepdims=True)
        upd = tmax > best
        return jnp.where(upd, tmax, best), jnp.where(upd, targ, bestj)

    best0 = jnp.full((half, 1), -jnp.inf, jnp.float32)
    bestj0 = jnp.zeros((half, 1), jnp.int32)
    _, dst = lax.fori_loop(0, nj, jbody, (best0, bestj0), unroll=False)
    dst_ref[pl.ds(0, half), :] = dst

    ni = half // _IB

    def ibody(t, carry):
        sums, cnt = carry
        i0 = pl.multiple_of(t * _IB, _IB)
        xat = get_a(i0, _IB)
        dst_t = dst_ref[pl.ds(i0, _IB), :]
        jio = lax.broadcasted_iota(jnp.int32, (_IB, half), 1)
        oh = (dst_t == jio).astype(jnp.bfloat16)
        hi = xat.astype(jnp.bfloat16)
        r1 = xat - hi.astype(jnp.float32)
        lo = r1.astype(jnp.bfloat16)
        lo2 = (r1 - lo.astype(jnp.float32)).astype(jnp.bfloat16)
        dn = (((1,), (0,)), ((), ()))
        part = (lax.dot_general(hi, oh, dn, preferred_element_type=jnp.float32)
                + lax.dot_general(lo, oh, dn, preferred_element_type=jnp.float32)
                + lax.dot_general(lo2, oh, dn, preferred_element_type=jnp.float32))
        sums = sums + part
        cnt = cnt + lax.dot_general(
            jnp.ones((1, _IB), jnp.bfloat16), oh, dn,
            preferred_element_type=jnp.float32)
        return sums, cnt

    sums0 = jnp.zeros((_C, half), jnp.float32)
    cnt0 = jnp.zeros((1, half), jnp.float32)
    sums, cnt = lax.fori_loop(0, ni, ibody, (sums0, cnt0), unroll=False)

    xb = get_b(0, half)
    return (xb + sums) / (1.0 + cnt)


def _body(sw_ref, dw_ref, x_ref, w_ref, g_ref, be_ref, out_ref,
          sp1_ref, sp2_ref, m_ref, dst_ref):
    b = pl.program_id(0)
    sw = sw_ref[0]
    dw = dw_ref[0]

    @pl.when(b == 0)
    def _():
        _fill_spatial(sp1_ref, _N // 2, _W, dw, 256)
        _fill_spatial(sp2_ref, _N // 4, 45, dw, 256)

    h1 = _N // 2

    def a1(c0, w):
        return x_ref[0, :, pl.ds(c0, w)]

    def b1(c0, w):
        return x_ref[0, :, pl.ds(pl.multiple_of(h1 + c0, _JB), w)]

    merged1 = _round(a1, b1, sp1_ref, dst_ref, h1, sw)
    m_ref[:, :] = merged1

    h2 = _N // 4

    def a2(c0, w):
        return m_ref[:, pl.ds(c0, w)]

    def b2(c0, w):
        return m_ref[:, pl.ds(pl.multiple_of(h2 + c0, _JB), w)]

    merged2 = _round(a2, b2, sp2_ref, dst_ref, h2, sw)

    y = lax.dot_general(w_ref[:, :].astype(jnp.bfloat16),
                        merged2.astype(jnp.bfloat16),
                        (((1,), (0,)), ((), ())),
                        preferred_element_type=jnp.float32)
    scale = g_ref[:, :] / jnp.sqrt(jnp.float32(1.0 + 1e-3))
    y = y * scale + be_ref[:, :]
    out_ref[0, :, :] = y * (1.0 / (1.0 + jnp.exp(-y)))


def kernel(x, sim_weight, dist_weight, conv_w, bn_gamma, bn_beta):
    xr = x.reshape(_B, _C, _N)
    sw = jnp.reshape(sim_weight, (1,)).astype(jnp.float32)
    dw = jnp.reshape(dist_weight, (1,)).astype(jnp.float32)
    g = jnp.reshape(bn_gamma, (_OUT, 1)).astype(jnp.float32)
    be = jnp.reshape(bn_beta, (_OUT, 1)).astype(jnp.float32)

    out = pl.pallas_call(
        _body,
        grid=(_B,),
        in_specs=[
            pl.BlockSpec(memory_space=pltpu.SMEM),
            pl.BlockSpec(memory_space=pltpu.SMEM),
            pl.BlockSpec((1, _C, _N), lambda b: (b, 0, 0)),
            pl.BlockSpec((_OUT, _C), lambda b: (0, 0)),
            pl.BlockSpec((_OUT, 1), lambda b: (0, 0)),
            pl.BlockSpec((_OUT, 1), lambda b: (0, 0)),
        ],
        out_specs=pl.BlockSpec((1, _OUT, _N // 4), lambda b: (b, 0, 0)),
        out_shape=jax.ShapeDtypeStruct((_B, _OUT, _N // 4), jnp.float32),
        scratch_shapes=[
            pltpu.VMEM((_N // 2, _N // 2), jnp.float32),
            pltpu.VMEM((_N // 4, _N // 4), jnp.float32),
            pltpu.VMEM((_C, _N // 2), jnp.float32),
            pltpu.VMEM((_N // 2, 1), jnp.int32),
        ],
        compiler_params=pltpu.CompilerParams(
            dimension_semantics=("arbitrary",)),
    )(sw, dw, xr, conv_w, g, be)
    return out.reshape(_B, _OUT, _H // 2, _W // 2)


# speedup vs baseline: 3.0962x; 1.0197x over previous
import jax
import jax.numpy as jnp
from jax import lax
from jax.experimental import pallas as pl
from jax.experimental.pallas import tpu as pltpu

_B, _C, _H, _W = 8, 128, 64, 64
_N = _H * _W
_OUT = 128
_JB = 1024
_IB = 1024
_BIG = 1 << 30


def _fill_spatial(sp_ref, half, width, dw, blk):
    nblk = half // blk

    def body(k, _):
        i0 = pl.multiple_of(k * blk, blk)
        ii = lax.broadcasted_iota(jnp.int32, (blk, 1), 0) + i0
        jj = lax.broadcasted_iota(jnp.int32, (1, half), 1) + half
        ra = (ii // width).astype(jnp.float32)
        ca = (ii % width).astype(jnp.float32)
        rb = (jj // width).astype(jnp.float32)
        cb = (jj % width).astype(jnp.float32)
        dr = ra - rb
        dc = ca - cb
        d2 = dr * dr + dc * dc
        sp_ref[pl.ds(i0, blk), :] = dw * (1.0 / (jnp.sqrt(d2) + 1e-6))
        return 0

    lax.fori_loop(0, nblk, body, 0, unroll=False)


def _round(get_a, get_b, sp_ref, dst_ref, half, sw):
    xa = get_a(0, half)
    na2 = jnp.sum(xa * xa, axis=0, keepdims=True)
    ma = (xa / jnp.sqrt(na2)).astype(jnp.bfloat16)

    nj = half // _JB

    def jbody(t, carry):
        best, bestj = carry
        j0 = pl.multiple_of(t * _JB, _JB)
        xbt = get_b(j0, _JB)
        nb2 = jnp.sum(xbt * xbt, axis=0, keepdims=True)
        mb = (xbt / jnp.sqrt(nb2)).astype(jnp.bfloat16)
        mm = lax.dot_general(ma, mb, (((0,), (0,)), ((), ())),
                             preferred_element_type=jnp.float32)
        s = sw * mm + sp_ref[:, pl.ds(j0, _JB)]
        tmax = jnp.max(s, axis=1, keepdims=True)
        jio = lax.broadcasted_iota(jnp.int32, (half, _JB), 1) + j0
        targ = jnp.min(jnp.where(s == tmax, jio, _BIG), axis=1,
                       keepdims=True)
        upd = tmax > best
        return jnp.where(upd, tmax, best), jnp.where(upd, targ, bestj)

    best0 = jnp.full((half, 1), -jnp.inf, jnp.float32)
    bestj0 = jnp.zeros((half, 1), jnp.int32)
    _, dst = lax.fori_loop(0, nj, jbody, (best0, bestj0), unroll=True)
    dst_ref[pl.ds(0, half), :] = dst

    ni = half // _IB

    def ibody(t, carry):
        sums, cnt = carry
        i0 = pl.multiple_of(t * _IB, _IB)
        xat = get_a(i0, _IB)
        dst_t = dst_ref[pl.ds(i0, _IB), :]
        jio = lax.broadcasted_iota(jnp.int32, (_IB, half), 1)
        oh = (dst_t == jio).astype(jnp.bfloat16)
        hi = xat.astype(jnp.bfloat16)
        r1 = xat - hi.astype(jnp.float32)
        lo = r1.astype(jnp.bfloat16)
        lo2 = (r1 - lo.astype(jnp.float32)).astype(jnp.bfloat16)
        dn = (((1,), (0,)), ((), ()))
        part = (lax.dot_general(hi, oh, dn, preferred_element_type=jnp.float32)
                + lax.dot_general(lo, oh, dn, preferred_element_type=jnp.float32)
                + lax.dot_general(lo2, oh, dn, preferred_element_type=jnp.float32))
        sums = sums + part
        cnt = cnt + lax.dot_general(
            jnp.ones((1, _IB), jnp.bfloat16), oh, dn,
            preferred_element_type=jnp.float32)
        return sums, cnt

    sums0 = jnp.zeros((_C, half), jnp.float32)
    cnt0 = jnp.zeros((1, half), jnp.float32)
    sums, cnt = lax.fori_loop(0, ni, ibody, (sums0, cnt0), unroll=True)

    xb = get_b(0, half)
    return (xb + sums) / (1.0 + cnt)


def _body(sw_ref, dw_ref, x_ref, w_ref, g_ref, be_ref, out_ref,
          sp1_ref, sp2_ref, m_ref, dst_ref):
    b = pl.program_id(0)
    sw = sw_ref[0]
    dw = dw_ref[0]

    @pl.when(b == 0)
    def _():
        _fill_spatial(sp1_ref, _N // 2, _W, dw, 256)
        _fill_spatial(sp2_ref, _N // 4, 45, dw, 256)

    h1 = _N // 2

    def a1(c0, w):
        return x_ref[0, :, pl.ds(c0, w)]

    def b1(c0, w):
        return x_ref[0, :, pl.ds(pl.multiple_of(h1 + c0, _JB), w)]

    merged1 = _round(a1, b1, sp1_ref, dst_ref, h1, sw)
    m_ref[:, :] = merged1

    h2 = _N // 4

    def a2(c0, w):
        return m_ref[:, pl.ds(c0, w)]

    def b2(c0, w):
        return m_ref[:, pl.ds(pl.multiple_of(h2 + c0, _JB), w)]

    merged2 = _round(a2, b2, sp2_ref, dst_ref, h2, sw)

    y = lax.dot_general(w_ref[:, :].astype(jnp.bfloat16),
                        merged2.astype(jnp.bfloat16),
                        (((1,), (0,)), ((), ())),
                        preferred_element_type=jnp.float32)
    scale = g_ref[:, :] / jnp.sqrt(jnp.float32(1.0 + 1e-3))
    y = y * scale + be_ref[:, :]
    out_ref[0, :, :] = y * (1.0 / (1.0 + jnp.exp(-y)))


def kernel(x, sim_weight, dist_weight, conv_w, bn_gamma, bn_beta):
    xr = x.reshape(_B, _C, _N)
    sw = jnp.reshape(sim_weight, (1,)).astype(jnp.float32)
    dw = jnp.reshape(dist_weight, (1,)).astype(jnp.float32)
    g = jnp.reshape(bn_gamma, (_OUT, 1)).astype(jnp.float32)
    be = jnp.reshape(bn_beta, (_OUT, 1)).astype(jnp.float32)

    out = pl.pallas_call(
        _body,
        grid=(_B,),
        in_specs=[
            pl.BlockSpec(memory_space=pltpu.SMEM),
            pl.BlockSpec(memory_space=pltpu.SMEM),
            pl.BlockSpec((1, _C, _N), lambda b: (b, 0, 0)),
            pl.BlockSpec((_OUT, _C), lambda b: (0, 0)),
            pl.BlockSpec((_OUT, 1), lambda b: (0, 0)),
            pl.BlockSpec((_OUT, 1), lambda b: (0, 0)),
        ],
        out_specs=pl.BlockSpec((1, _OUT, _N // 4), lambda b: (b, 0, 0)),
        out_shape=jax.ShapeDtypeStruct((_B, _OUT, _N // 4), jnp.float32),
        scratch_shapes=[
            pltpu.VMEM((_N // 2, _N // 2), jnp.float32),
            pltpu.VMEM((_N // 4, _N // 4), jnp.float32),
            pltpu.VMEM((_C, _N // 2), jnp.float32),
            pltpu.VMEM((_N // 2, 1), jnp.int32),
        ],
        compiler_params=pltpu.CompilerParams(
            dimension_semantics=("arbitrary",)),
    )(sw, dw, xr, conv_w, g, be)
    return out.reshape(_B, _OUT, _H // 2, _W // 2)


# speedup vs baseline: 3.2422x; 1.0472x over previous
import jax
import jax.numpy as jnp
from jax import lax
from jax.experimental import pallas as pl
from jax.experimental.pallas import tpu as pltpu

_B, _C, _H, _W = 8, 128, 64, 64
_N = _H * _W
_OUT = 128
_JB = 1024
_IB = 1024
_BIG = 1 << 30


def _fill_spatial(sp_ref, half, width, dw, blk):
    nblk = half // blk

    def body(k, _):
        i0 = pl.multiple_of(k * blk, blk)
        ii = lax.broadcasted_iota(jnp.int32, (blk, 1), 0) + i0
        jj = lax.broadcasted_iota(jnp.int32, (1, half), 1) + half
        ra = (ii // width).astype(jnp.float32)
        ca = (ii % width).astype(jnp.float32)
        rb = (jj // width).astype(jnp.float32)
        cb = (jj % width).astype(jnp.float32)
        dr = ra - rb
        dc = ca - cb
        d2 = dr * dr + dc * dc
        sp_ref[pl.ds(i0, blk), :] = dw * (1.0 / (jnp.sqrt(d2) + 1e-6))
        return 0

    lax.fori_loop(0, nblk, body, 0, unroll=False)


def _round(get_a, get_b, sp_ref, dst_ref, half, sw):
    xa = get_a(0, half)
    na2 = jnp.sum(xa * xa, axis=0, keepdims=True)
    ma = (xa / jnp.sqrt(na2)).astype(jnp.bfloat16)

    nj = half // _JB

    def jbody(t, carry):
        best, bestj = carry
        j0 = pl.multiple_of(t * _JB, _JB)
        xbt = get_b(j0, _JB)
        nb2 = jnp.sum(xbt * xbt, axis=0, keepdims=True)
        mb = (xbt / jnp.sqrt(nb2)).astype(jnp.bfloat16)
        mm = lax.dot_general(ma, mb, (((0,), (0,)), ((), ())),
                             preferred_element_type=jnp.float32)
        s = sw * mm + sp_ref[:, pl.ds(j0, _JB)]
        tmax = jnp.max(s, axis=1, keepdims=True)
        jio = lax.broadcasted_iota(jnp.int32, (half, _JB), 1) + j0
        targ = jnp.min(jnp.where(s == tmax, jio, _BIG), axis=1,
                       keepdims=True)
        upd = tmax > best
        return jnp.where(upd, tmax, best), jnp.where(upd, targ, bestj)

    best0 = jnp.full((half, 1), -jnp.inf, jnp.float32)
    bestj0 = jnp.zeros((half, 1), jnp.int32)
    _, dst = lax.fori_loop(0, nj, jbody, (best0, bestj0), unroll=True)
    dst_ref[pl.ds(0, half), :] = dst

    ni = half // _IB

    def ibody(t, carry):
        acc = carry
        i0 = pl.multiple_of(t * _IB, _IB)
        xat = get_a(i0, _IB)
        dst_t = dst_ref[pl.ds(i0, _IB), :]
        jio = lax.broadcasted_iota(jnp.int32, (_IB, half), 1)
        oh = (dst_t == jio).astype(jnp.bfloat16)
        xe = jnp.concatenate([xat, jnp.ones((1, _IB), jnp.float32)], axis=0)
        hi = xe.astype(jnp.bfloat16)
        r1 = xe - hi.astype(jnp.float32)
        lo = r1.astype(jnp.bfloat16)
        lo2 = (r1 - lo.astype(jnp.float32)).astype(jnp.bfloat16)
        dn = (((1,), (0,)), ((), ()))
        part = (lax.dot_general(hi, oh, dn, preferred_element_type=jnp.float32)
                + lax.dot_general(lo, oh, dn, preferred_element_type=jnp.float32)
                + lax.dot_general(lo2, oh, dn, preferred_element_type=jnp.float32))
        return acc + part

    acc0 = jnp.zeros((_C + 1, half), jnp.float32)
    acc = lax.fori_loop(0, ni, ibody, acc0, unroll=True)
    sums = acc[:_C, :]
    cnt = acc[_C:, :]

    xb = get_b(0, half)
    return (xb + sums) / (1.0 + cnt)


def _body(sw_ref, dw_ref, x_ref, w_ref, g_ref, be_ref, out_ref,
          sp1_ref, sp2_ref, m_ref, dst_ref):
    b = pl.program_id(0)
    sw = sw_ref[0]
    dw = dw_ref[0]

    @pl.when(b == 0)
    def _():
        _fill_spatial(sp1_ref, _N // 2, _W, dw, 256)
        _fill_spatial(sp2_ref, _N // 4, 45, dw, 256)

    h1 = _N // 2

    def a1(c0, w):
        return x_ref[0, :, pl.ds(c0, w)]

    def b1(c0, w):
        return x_ref[0, :, pl.ds(pl.multiple_of(h1 + c0, _JB), w)]

    merged1 = _round(a1, b1, sp1_ref, dst_ref, h1, sw)
    m_ref[:, :] = merged1

    h2 = _N // 4

    def a2(c0, w):
        return m_ref[:, pl.ds(c0, w)]

    def b2(c0, w):
        return m_ref[:, pl.ds(pl.multiple_of(h2 + c0, _JB), w)]

    merged2 = _round(a2, b2, sp2_ref, dst_ref, h2, sw)

    y = lax.dot_general(w_ref[:, :].astype(jnp.bfloat16),
                        merged2.astype(jnp.bfloat16),
                        (((1,), (0,)), ((), ())),
                        preferred_element_type=jnp.float32)
    scale = g_ref[:, :] / jnp.sqrt(jnp.float32(1.0 + 1e-3))
    y = y * scale + be_ref[:, :]
    out_ref[0, :, :] = y * (1.0 / (1.0 + jnp.exp(-y)))


def kernel(x, sim_weight, dist_weight, conv_w, bn_gamma, bn_beta):
    xr = x.reshape(_B, _C, _N)
    sw = jnp.reshape(sim_weight, (1,)).astype(jnp.float32)
    dw = jnp.reshape(dist_weight, (1,)).astype(jnp.float32)
    g = jnp.reshape(bn_gamma, (_OUT, 1)).astype(jnp.float32)
    be = jnp.reshape(bn_beta, (_OUT, 1)).astype(jnp.float32)

    out = pl.pallas_call(
        _body,
        grid=(_B,),
        in_specs=[
            pl.BlockSpec(memory_space=pltpu.SMEM),
            pl.BlockSpec(memory_space=pltpu.SMEM),
            pl.BlockSpec((1, _C, _N), lambda b: (b, 0, 0)),
            pl.BlockSpec((_OUT, _C), lambda b: (0, 0)),
            pl.BlockSpec((_OUT, 1), lambda b: (0, 0)),
            pl.BlockSpec((_OUT, 1), lambda b: (0, 0)),
        ],
        out_specs=pl.BlockSpec((1, _OUT, _N // 4), lambda b: (b, 0, 0)),
        out_shape=jax.ShapeDtypeStruct((_B, _OUT, _N // 4), jnp.float32),
        scratch_shapes=[
            pltpu.VMEM((_N // 2, _N // 2), jnp.float32),
            pltpu.VMEM((_N // 4, _N // 4), jnp.float32),
            pltpu.VMEM((_C, _N // 2), jnp.float32),
            pltpu.VMEM((_N // 2, 1), jnp.int32),
        ],
        compiler_params=pltpu.CompilerParams(
            dimension_semantics=("arbitrary",)),
    )(sw, dw, xr, conv_w, g, be)
    return out.reshape(_B, _OUT, _H // 2, _W // 2)
